# baseline (speedup 1.0000x reference)
import functools

import jax
import jax.numpy as jnp
from jax.experimental import pallas as pl

_N = 10000
_BR = 400


def _proj_body(x_ref, w1_ref, s1_ref):
    s1 = jnp.dot(x_ref[...], w1_ref[...], preferred_element_type=jnp.float32)
    s1_ref[...] = s1.astype(jnp.bfloat16)


def _layer1_body(adj_ref, s1_ref, w2_ref, s2_ref):
    a = adj_ref[...].astype(jnp.bfloat16)
    h = jnp.dot(a, s1_ref[...], preferred_element_type=jnp.float32)
    x1 = jnp.where(h >= 0, h, 0.01 * h)
    s2 = jnp.dot(x1.astype(jnp.bfloat16), w2_ref[...],
                 preferred_element_type=jnp.float32)
    s2_ref[...] = s2.astype(jnp.bfloat16)


def _layer2_body(adj_ref, s2_ref, wout_ref, x3_ref, y_ref):
    a = adj_ref[...].astype(jnp.bfloat16)
    x3 = jnp.dot(a, s2_ref[...], preferred_element_type=jnp.float32)
    x3_ref[...] = x3
    logits = jnp.dot(x3.astype(jnp.bfloat16), wout_ref[...],
                     preferred_element_type=jnp.float32)
    y_ref[...] = jax.nn.sigmoid(logits)


@functools.partial(jax.jit, static_argnames=())
def kernel(x, adj, W1, W2, W_out):
    n, nfeat = x.shape
    nhid = W1.shape[1]
    nclass = W_out.shape[1]
    grid = (n // _BR,)

    s1 = pl.pallas_call(
        _proj_body,
        out_shape=jax.ShapeDtypeStruct((n, nhid), jnp.bfloat16),
    )(x, W1)

    w2_b = W2.astype(jnp.bfloat16)
    s2 = pl.pallas_call(
        _layer1_body,
        grid=grid,
        in_specs=[
            pl.BlockSpec((_BR, n), lambda i: (i, 0)),
            pl.BlockSpec((n, nhid), lambda i: (0, 0)),
            pl.BlockSpec((nhid, nhid), lambda i: (0, 0)),
        ],
        out_specs=pl.BlockSpec((_BR, nhid), lambda i: (i, 0)),
        out_shape=jax.ShapeDtypeStruct((n, nhid), jnp.bfloat16),
    )(adj, s1, w2_b)

    wout_b = W_out.astype(jnp.bfloat16)
    x3, y = pl.pallas_call(
        _layer2_body,
        grid=grid,
        in_specs=[
            pl.BlockSpec((_BR, n), lambda i: (i, 0)),
            pl.BlockSpec((n, nhid), lambda i: (0, 0)),
            pl.BlockSpec((nhid, nclass), lambda i: (0, 0)),
        ],
        out_specs=[
            pl.BlockSpec((_BR, nhid), lambda i: (i, 0)),
            pl.BlockSpec((_BR, nclass), lambda i: (i, 0)),
        ],
        out_shape=[
            jax.ShapeDtypeStruct((n, nhid), jnp.float32),
            jax.ShapeDtypeStruct((n, nclass), jnp.float32),
        ],
    )(adj, s2, wout_b)

    return (y, x3)
